# baseline (speedup 1.0000x reference)
import functools

import jax
import jax.numpy as jnp
from jax import lax
from jax.experimental import pallas as pl
from jax.experimental.pallas import tpu as pltpu
from jax.experimental.pallas import tpu_sc as plsc

N = 10000
D = 128
E = 320000
EPS = 1e-5

NC = 2
NS = 16
NW = NC * NS
LANES = 16

NP = 10240
RPT = NP // NS
K = 80
EPT = E // NW
NCHUNK = EPT // K
ZCH = 128

_MESH = dict(core_axis_name="c", subcore_axis_name="s")


def _sc_degrees(idx2):
    @functools.partial(
        pl.kernel,
        out_type=jax.ShapeDtypeStruct((2, NW, N), jnp.float32),
        mesh=plsc.VectorSubcoreMesh(**_MESH),
        compiler_params=pltpu.CompilerParams(needs_layout_passes=False),
        scratch_types=[
            pltpu.VMEM((EPT,), jnp.int32),
            pltpu.VMEM((N,), jnp.float32),
        ],
    )
    def deg_kernel(idx_hbm, out_hbm, idx_v, deg_v):
        c = lax.axis_index("c")
        s = lax.axis_index("s")
        wid = c * NS + s
        ones = jnp.ones((LANES,), jnp.float32)
        zeros = jnp.zeros((LANES,), jnp.float32)
        U = 5
        for which in range(2):
            pltpu.sync_copy(idx_hbm.at[which, wid], idx_v)

            def zbody(i, _):
                for u in range(U):
                    deg_v[pl.ds((i * U + u) * LANES, LANES)] = zeros
                return 0

            lax.fori_loop(0, N // LANES // U, zbody, 0)

            def abody(i, _):
                for u in range(U):
                    idx = idx_v[pl.ds((i * U + u) * LANES, LANES)]
                    plsc.addupdate_scatter(deg_v, [idx], ones)
                return 0

            lax.fori_loop(0, EPT // LANES // U, abody, 0)
            pltpu.sync_copy(deg_v, out_hbm.at[which, wid])

    return deg_kernel(idx2)


def _sc_aggregate(y, srcc, dstc):
    @functools.partial(
        pl.kernel,
        out_type=jax.ShapeDtypeStruct((NC, NP, D), jnp.float32),
        mesh=plsc.VectorSubcoreMesh(**_MESH),
        compiler_params=pltpu.CompilerParams(needs_layout_passes=False),
        scratch_types=[
            pltpu.VMEM((EPT,), jnp.int32),
            pltpu.VMEM((NCHUNK, K), jnp.int32),
            pltpu.VMEM((K, D), jnp.float32),
            pltpu.VMEM((K, D), jnp.float32),
            pltpu.VMEM_SHARED((NP, D), jnp.float32),
            pltpu.SemaphoreType.DMA,
            pltpu.SemaphoreType.DMA,
        ],
    )
    def agg_kernel(y_hbm, src_hbm, dst_hbm, out_hbm,
                   src_v, dst_v, buf0, buf1, acc, sem0, sem1):
        c = lax.axis_index("c")
        s = lax.axis_index("s")
        wid = c * NS + s

        pltpu.sync_copy(src_hbm.at[wid], src_v)
        pltpu.sync_copy(dst_hbm.at[wid], dst_v)

        zeros = jnp.zeros((LANES,), jnp.float32)

        def zb(i, _):
            for j in range(D // LANES):
                buf0[i, pl.ds(j * LANES, LANES)] = zeros
            return 0

        lax.fori_loop(0, K, zb, 0)

        for z in range(RPT // K):
            pltpu.sync_copy(buf0, acc.at[pl.ds(s * RPT + z * K, K)])
        plsc.subcore_barrier()

        bufs = (buf0, buf1)
        sems = (sem0, sem1)
        KH = K // 2

        def gstart(jj, b):
            pltpu.async_copy(
                y_hbm.at[src_v.at[pl.ds(jj * K, KH)]],
                bufs[b].at[pl.ds(0, KH)], sems[b])
            pltpu.async_copy(
                y_hbm.at[src_v.at[pl.ds(jj * K + KH, KH)]],
                bufs[b].at[pl.ds(KH, KH)], sems[b])

        def gwait(jj, b):
            pltpu.make_async_copy(
                y_hbm.at[src_v.at[pl.ds(jj * K, KH)]],
                bufs[b].at[pl.ds(0, KH)], sems[b]).wait()
            pltpu.make_async_copy(
                y_hbm.at[src_v.at[pl.ds(jj * K + KH, KH)]],
                bufs[b].at[pl.ds(KH, KH)], sems[b]).wait()

        gstart(0, 0)

        def body(j, _):
            for b in range(2):
                jj = j * 2 + b
                nxt = jj + 1
                @pl.when(nxt < NCHUNK)
                def _():
                    gstart(nxt, (b + 1) % 2)
                gwait(jj, b)
                pltpu.sync_copy(bufs[b], acc.at[dst_v.at[jj]], add=True)
            return 0

        lax.fori_loop(0, NCHUNK // 2, body, 0)
        if NCHUNK % 2:
            jj = NCHUNK - 1
            gwait(jj, jj % 2)
            pltpu.sync_copy(bufs[jj % 2], acc.at[dst_v.at[jj]], add=True)

        plsc.subcore_barrier()
        for z in range(RPT // ZCH):
            rows = pl.ds(s * RPT + z * ZCH, ZCH)
            pltpu.sync_copy(acc.at[rows], out_hbm.at[c, rows])

    return agg_kernel(y, srcc, dstc)


def _tc_mm_body(x_ref, w_ref, xw_ref):
    xw_ref[...] = jnp.dot(x_ref[...], w_ref[...],
                          preferred_element_type=jnp.float32)


def _tc_mm(x, W1):
    return pl.pallas_call(
        _tc_mm_body,
        out_shape=jax.ShapeDtypeStruct((N, D), jnp.float32),
    )(x, W1)


def _pad_rows(y):
    return jnp.concatenate(
        [y, jnp.zeros((NP - N, D), jnp.float32)], axis=0)


def _tc_scale_body(degp_ref, xw_ref, y_ref, ns_ref, nd_ref):
    deg = jnp.sum(degp_ref[...], axis=1)
    ns = lax.rsqrt(jnp.maximum(deg[0], 1.0))
    nd = lax.rsqrt(jnp.maximum(deg[1], 1.0))
    ns_ref[...] = ns[None, :]
    nd_ref[...] = nd[None, :]
    y_ref[...] = _pad_rows(xw_ref[...] * ns[:, None])


def _tc_scale(degp, xw):
    return pl.pallas_call(
        _tc_scale_body,
        out_shape=(
            jax.ShapeDtypeStruct((NP, D), jnp.float32),
            jax.ShapeDtypeStruct((1, N), jnp.float32),
            jax.ShapeDtypeStruct((1, N), jnp.float32),
        ),
    )(degp, xw)


def _tc_mid_body(p_ref, nd_ref, b1_ref, g_ref, be_ref, w2_ref, ns_ref,
                 y2_ref):
    p = p_ref[...]
    h = (p[0, :N] + p[1, :N]) * nd_ref[0][:, None] + b1_ref[0][None, :]
    mean = jnp.mean(h, axis=0)
    cent = h - mean[None, :]
    var = jnp.mean(cent * cent, axis=0)
    hb = cent * lax.rsqrt(var + EPS)[None, :] * g_ref[0][None, :] \
        + be_ref[0][None, :]
    r = jnp.maximum(hb, 0.0)
    rw = jnp.dot(r, w2_ref[...], preferred_element_type=jnp.float32)
    y2_ref[...] = _pad_rows(rw * ns_ref[0][:, None])


def _tc_mid(p, nd, b1, gamma, beta, W2, ns):
    return pl.pallas_call(
        _tc_mid_body,
        out_shape=jax.ShapeDtypeStruct((NP, D), jnp.float32),
    )(p, nd, b1, gamma, beta, W2, ns)


def _tc_out_body(p_ref, nd_ref, b2_ref, o_ref):
    p = p_ref[...]
    o_ref[...] = (p[0, :N] + p[1, :N]) * nd_ref[0][:, None] \
        + b2_ref[0][None, :]


def _tc_out(p, nd, b2):
    return pl.pallas_call(
        _tc_out_body,
        out_shape=jax.ShapeDtypeStruct((N, D), jnp.float32),
    )(p, nd, b2)


@jax.jit
def _run(x, edge_index, W1, b1, gamma, beta, W2, b2):
    src = edge_index[0].astype(jnp.int32)
    dst = edge_index[1].astype(jnp.int32)

    idx2 = jnp.stack([src, dst]).reshape(2, NW, EPT)
    srcc = src.reshape(NW, EPT)
    dstc = dst.reshape(NW, NCHUNK, K)

    degp = _sc_degrees(idx2)
    xw = _tc_mm(x, W1)
    y1, ns, nd = _tc_scale(degp, xw)
    p1 = _sc_aggregate(y1, srcc, dstc)
    y2 = _tc_mid(p1, nd, b1.reshape(1, D), gamma.reshape(1, D),
                 beta.reshape(1, D), W2, ns)
    p2 = _sc_aggregate(y2, srcc, dstc)
    return _tc_out(p2, nd, b2.reshape(1, D))


def kernel(x, edge_index, W1, b1, gamma, beta, W2, b2):
    return _run(x, edge_index, W1, b1, gamma, beta, W2, b2)


# speedup vs baseline: 1.0009x; 1.0009x over previous
import functools

import jax
import jax.numpy as jnp
from jax import lax
from jax.experimental import pallas as pl
from jax.experimental.pallas import tpu as pltpu
from jax.experimental.pallas import tpu_sc as plsc

N = 10000
D = 128
E = 320000
EPS = 1e-5

NC = 2
NS = 16
NW = NC * NS
LANES = 16

NP = 10240
RPT = NP // NS
K = 80
EPT = E // NW
NCHUNK = EPT // K
ZCH = 128

_MESH = dict(core_axis_name="c", subcore_axis_name="s")


def _sc_degrees(idx2):
    @functools.partial(
        pl.kernel,
        out_type=jax.ShapeDtypeStruct((2, NW, N), jnp.float32),
        mesh=plsc.VectorSubcoreMesh(**_MESH),
        compiler_params=pltpu.CompilerParams(needs_layout_passes=False),
        scratch_types=[
            pltpu.VMEM((EPT,), jnp.int32),
            pltpu.VMEM((N,), jnp.float32),
        ],
    )
    def deg_kernel(idx_hbm, out_hbm, idx_v, deg_v):
        c = lax.axis_index("c")
        s = lax.axis_index("s")
        wid = c * NS + s
        ones = jnp.ones((LANES,), jnp.float32)
        zeros = jnp.zeros((LANES,), jnp.float32)
        U = 5
        for which in range(2):
            pltpu.sync_copy(idx_hbm.at[which, wid], idx_v)

            def zbody(i, _):
                for u in range(U):
                    deg_v[pl.ds((i * U + u) * LANES, LANES)] = zeros
                return 0

            lax.fori_loop(0, N // LANES // U, zbody, 0)

            def abody(i, _):
                for u in range(U):
                    idx = idx_v[pl.ds((i * U + u) * LANES, LANES)]
                    plsc.addupdate_scatter(deg_v, [idx], ones)
                return 0

            lax.fori_loop(0, EPT // LANES // U, abody, 0)
            pltpu.sync_copy(deg_v, out_hbm.at[which, wid])

    return deg_kernel(idx2)


def _sc_aggregate(y, srcc, dstc):
    @functools.partial(
        pl.kernel,
        out_type=jax.ShapeDtypeStruct((NC, NP, D), jnp.float32),
        mesh=plsc.VectorSubcoreMesh(**_MESH),
        compiler_params=pltpu.CompilerParams(needs_layout_passes=False),
        scratch_types=[
            pltpu.VMEM((EPT,), jnp.int32),
            pltpu.VMEM((NCHUNK, K), jnp.int32),
            pltpu.VMEM((K, D), jnp.float32),
            pltpu.VMEM((K, D), jnp.float32),
            pltpu.VMEM_SHARED((NP, D), jnp.float32),
            pltpu.SemaphoreType.DMA,
            pltpu.SemaphoreType.DMA,
        ],
    )
    def agg_kernel(y_hbm, src_hbm, dst_hbm, out_hbm,
                   src_v, dst_v, buf0, buf1, acc, sem0, sem1):
        c = lax.axis_index("c")
        s = lax.axis_index("s")
        wid = c * NS + s

        pltpu.sync_copy(src_hbm.at[wid], src_v)
        pltpu.sync_copy(dst_hbm.at[wid], dst_v)

        zeros = jnp.zeros((LANES,), jnp.float32)

        def zb(i, _):
            for j in range(D // LANES):
                buf0[i, pl.ds(j * LANES, LANES)] = zeros
            return 0

        lax.fori_loop(0, K, zb, 0)

        for z in range(RPT // K):
            pltpu.sync_copy(buf0, acc.at[pl.ds(s * RPT + z * K, K)])
        plsc.subcore_barrier()

        bufs = (buf0, buf1)
        sems = (sem0, sem1)
        pltpu.async_copy(y_hbm.at[src_v.at[pl.ds(0, K)]], buf0, sem0)

        def body(j, _):
            for b in range(2):
                jj = j * 2 + b
                nxt = jj + 1
                @pl.when(nxt < NCHUNK)
                def _():
                    pltpu.async_copy(
                        y_hbm.at[src_v.at[pl.ds(nxt * K, K)]],
                        bufs[(b + 1) % 2], sems[(b + 1) % 2])
                pltpu.make_async_copy(
                    y_hbm.at[src_v.at[pl.ds(jj * K, K)]], bufs[b],
                    sems[b]).wait()
                pltpu.sync_copy(bufs[b], acc.at[dst_v.at[jj]], add=True)
            return 0

        lax.fori_loop(0, NCHUNK // 2, body, 0)
        if NCHUNK % 2:
            jj = NCHUNK - 1
            pltpu.make_async_copy(
                y_hbm.at[src_v.at[pl.ds(jj * K, K)]], bufs[jj % 2],
                sems[jj % 2]).wait()
            pltpu.sync_copy(bufs[jj % 2], acc.at[dst_v.at[jj]], add=True)

        plsc.subcore_barrier()
        for z in range(RPT // ZCH):
            rows = pl.ds(s * RPT + z * ZCH, ZCH)
            pltpu.sync_copy(acc.at[rows], out_hbm.at[c, rows])

    return agg_kernel(y, srcc, dstc)


def _tc_mm_body(x_ref, w_ref, xw_ref):
    xw_ref[...] = jnp.dot(x_ref[...], w_ref[...],
                          preferred_element_type=jnp.float32)


def _tc_mm(x, W1):
    return pl.pallas_call(
        _tc_mm_body,
        out_shape=jax.ShapeDtypeStruct((N, D), jnp.float32),
    )(x, W1)


def _pad_rows(y):
    return jnp.concatenate(
        [y, jnp.zeros((NP - N, D), jnp.float32)], axis=0)


def _tc_scale_body(degp_ref, xw_ref, y_ref, ns_ref, nd_ref):
    deg = jnp.sum(degp_ref[...], axis=1)
    ns = lax.rsqrt(jnp.maximum(deg[0], 1.0))
    nd = lax.rsqrt(jnp.maximum(deg[1], 1.0))
    ns_ref[...] = ns[None, :]
    nd_ref[...] = nd[None, :]
    y_ref[...] = _pad_rows(xw_ref[...] * ns[:, None])


def _tc_scale(degp, xw):
    return pl.pallas_call(
        _tc_scale_body,
        out_shape=(
            jax.ShapeDtypeStruct((NP, D), jnp.float32),
            jax.ShapeDtypeStruct((1, N), jnp.float32),
            jax.ShapeDtypeStruct((1, N), jnp.float32),
        ),
    )(degp, xw)


def _tc_mid_body(p_ref, nd_ref, b1_ref, g_ref, be_ref, w2_ref, ns_ref,
                 y2_ref):
    p = p_ref[...]
    h = (p[0, :N] + p[1, :N]) * nd_ref[0][:, None] + b1_ref[0][None, :]
    mean = jnp.mean(h, axis=0)
    cent = h - mean[None, :]
    var = jnp.mean(cent * cent, axis=0)
    hb = cent * lax.rsqrt(var + EPS)[None, :] * g_ref[0][None, :] \
        + be_ref[0][None, :]
    r = jnp.maximum(hb, 0.0)
    rw = jnp.dot(r, w2_ref[...], preferred_element_type=jnp.float32)
    y2_ref[...] = _pad_rows(rw * ns_ref[0][:, None])


def _tc_mid(p, nd, b1, gamma, beta, W2, ns):
    return pl.pallas_call(
        _tc_mid_body,
        out_shape=jax.ShapeDtypeStruct((NP, D), jnp.float32),
    )(p, nd, b1, gamma, beta, W2, ns)


def _tc_out_body(p_ref, nd_ref, b2_ref, o_ref):
    p = p_ref[...]
    o_ref[...] = (p[0, :N] + p[1, :N]) * nd_ref[0][:, None] \
        + b2_ref[0][None, :]


def _tc_out(p, nd, b2):
    return pl.pallas_call(
        _tc_out_body,
        out_shape=jax.ShapeDtypeStruct((N, D), jnp.float32),
    )(p, nd, b2)


@jax.jit
def _run(x, edge_index, W1, b1, gamma, beta, W2, b2):
    src = edge_index[0].astype(jnp.int32)
    dst = edge_index[1].astype(jnp.int32)

    idx2 = jnp.stack([src, dst]).reshape(2, NW, EPT)
    srcc = src.reshape(NW, EPT)
    dstc = dst.reshape(NW, NCHUNK, K)

    degp = _sc_degrees(idx2)
    xw = _tc_mm(x, W1)
    y1, ns, nd = _tc_scale(degp, xw)
    p1 = _sc_aggregate(y1, srcc, dstc)
    y2 = _tc_mid(p1, nd, b1.reshape(1, D), gamma.reshape(1, D),
                 beta.reshape(1, D), W2, ns)
    p2 = _sc_aggregate(y2, srcc, dstc)
    return _tc_out(p2, nd, b2.reshape(1, D))


def kernel(x, edge_index, W1, b1, gamma, beta, W2, b2):
    return _run(x, edge_index, W1, b1, gamma, beta, W2, b2)


# speedup vs baseline: 1.0176x; 1.0167x over previous
import functools

import jax
import jax.numpy as jnp
from jax import lax
from jax.experimental import pallas as pl
from jax.experimental.pallas import tpu as pltpu
from jax.experimental.pallas import tpu_sc as plsc

N = 10000
D = 128
E = 320000
EPS = 1e-5

NC = 2
NS = 16
NW = NC * NS
LANES = 16

NP = 10240
RPT = NP // NS
K = 80
EPT = E // NW
NCHUNK = EPT // K
ZCH = 128

_MESH = dict(core_axis_name="c", subcore_axis_name="s")


def _sc_degrees(idx2):
    @functools.partial(
        pl.kernel,
        out_type=jax.ShapeDtypeStruct((2, NW, N), jnp.float32),
        mesh=plsc.VectorSubcoreMesh(**_MESH),
        compiler_params=pltpu.CompilerParams(needs_layout_passes=False),
        scratch_types=[
            pltpu.VMEM((EPT,), jnp.int32),
            pltpu.VMEM((N,), jnp.float32),
        ],
    )
    def deg_kernel(idx_hbm, out_hbm, idx_v, deg_v):
        c = lax.axis_index("c")
        s = lax.axis_index("s")
        wid = c * NS + s
        ones = jnp.ones((LANES,), jnp.float32)
        zeros = jnp.zeros((LANES,), jnp.float32)
        U = 5
        for which in range(2):
            pltpu.sync_copy(idx_hbm.at[which, wid], idx_v)

            def zbody(i, _):
                for u in range(U):
                    deg_v[pl.ds((i * U + u) * LANES, LANES)] = zeros
                return 0

            lax.fori_loop(0, N // LANES // U, zbody, 0)

            def abody(i, _):
                for u in range(U):
                    idx = idx_v[pl.ds((i * U + u) * LANES, LANES)]
                    plsc.addupdate_scatter(deg_v, [idx], ones)
                return 0

            lax.fori_loop(0, EPT // LANES // U, abody, 0)
            pltpu.sync_copy(deg_v, out_hbm.at[which, wid])

    return deg_kernel(idx2)


def _sc_aggregate(y, srcc, dstc):
    @functools.partial(
        pl.kernel,
        out_type=jax.ShapeDtypeStruct((NC, NP, D), jnp.float32),
        mesh=plsc.VectorSubcoreMesh(**_MESH),
        compiler_params=pltpu.CompilerParams(needs_layout_passes=False),
        scratch_types=[
            pltpu.VMEM((EPT,), jnp.int32),
            pltpu.VMEM((NCHUNK, K), jnp.int32),
            pltpu.VMEM((K, D), jnp.float32),
            pltpu.VMEM((K, D), jnp.float32),
            pltpu.VMEM_SHARED((NP, D), jnp.float32),
            pltpu.SemaphoreType.DMA,
            pltpu.SemaphoreType.DMA,
        ],
    )
    def agg_kernel(y_hbm, src_hbm, dst_hbm, out_hbm,
                   src_v, dst_v, buf0, buf1, acc, sem0, sem1):
        c = lax.axis_index("c")
        s = lax.axis_index("s")
        wid = c * NS + s

        pltpu.async_copy(src_hbm.at[wid], src_v, sem0)
        pltpu.async_copy(dst_hbm.at[wid], dst_v, sem1)

        zeros = jnp.zeros((LANES,), jnp.float32)

        def zb(i, _):
            for j in range(D // LANES):
                buf0[i, pl.ds(j * LANES, LANES)] = zeros
            return 0

        lax.fori_loop(0, K, zb, 0)

        for z in range(RPT // K):
            pltpu.sync_copy(buf0, acc.at[pl.ds(s * RPT + z * K, K)])
        pltpu.make_async_copy(src_hbm.at[wid], src_v, sem0).wait()
        pltpu.make_async_copy(dst_hbm.at[wid], dst_v, sem1).wait()
        plsc.subcore_barrier()

        bufs = (buf0, buf1)
        sems = (sem0, sem1)
        pltpu.async_copy(y_hbm.at[src_v.at[pl.ds(0, K)]], buf0, sem0)

        def body(j, _):
            for b in range(2):
                jj = j * 2 + b
                nxt = jj + 1
                @pl.when(nxt < NCHUNK)
                def _():
                    pltpu.async_copy(
                        y_hbm.at[src_v.at[pl.ds(nxt * K, K)]],
                        bufs[(b + 1) % 2], sems[(b + 1) % 2])
                pltpu.make_async_copy(
                    y_hbm.at[src_v.at[pl.ds(jj * K, K)]], bufs[b],
                    sems[b]).wait()
                pltpu.sync_copy(bufs[b], acc.at[dst_v.at[jj]], add=True)
            return 0

        lax.fori_loop(0, NCHUNK // 2, body, 0)
        if NCHUNK % 2:
            jj = NCHUNK - 1
            pltpu.make_async_copy(
                y_hbm.at[src_v.at[pl.ds(jj * K, K)]], bufs[jj % 2],
                sems[jj % 2]).wait()
            pltpu.sync_copy(bufs[jj % 2], acc.at[dst_v.at[jj]], add=True)

        plsc.subcore_barrier()
        for z in range(RPT // ZCH):
            rows = pl.ds(s * RPT + z * ZCH, ZCH)
            pltpu.sync_copy(acc.at[rows], out_hbm.at[c, rows])

    return agg_kernel(y, srcc, dstc)


def _tc_mm_body(x_ref, w_ref, xw_ref):
    xw_ref[...] = jnp.dot(x_ref[...], w_ref[...],
                          preferred_element_type=jnp.float32)


def _tc_mm(x, W1):
    return pl.pallas_call(
        _tc_mm_body,
        out_shape=jax.ShapeDtypeStruct((N, D), jnp.float32),
    )(x, W1)


def _pad_rows(y):
    return jnp.concatenate(
        [y, jnp.zeros((NP - N, D), jnp.float32)], axis=0)


def _tc_scale_body(degp_ref, xw_ref, y_ref, ns_ref, nd_ref):
    deg = jnp.sum(degp_ref[...], axis=1)
    ns = lax.rsqrt(jnp.maximum(deg[0], 1.0))
    nd = lax.rsqrt(jnp.maximum(deg[1], 1.0))
    ns_ref[...] = ns[None, :]
    nd_ref[...] = nd[None, :]
    y_ref[...] = _pad_rows(xw_ref[...] * ns[:, None])


def _tc_scale(degp, xw):
    return pl.pallas_call(
        _tc_scale_body,
        out_shape=(
            jax.ShapeDtypeStruct((NP, D), jnp.float32),
            jax.ShapeDtypeStruct((1, N), jnp.float32),
            jax.ShapeDtypeStruct((1, N), jnp.float32),
        ),
    )(degp, xw)


def _tc_mid_body(p_ref, nd_ref, b1_ref, g_ref, be_ref, w2_ref, ns_ref,
                 y2_ref):
    p = p_ref[...]
    h = (p[0, :N] + p[1, :N]) * nd_ref[0][:, None] + b1_ref[0][None, :]
    mean = jnp.mean(h, axis=0)
    cent = h - mean[None, :]
    var = jnp.mean(cent * cent, axis=0)
    hb = cent * lax.rsqrt(var + EPS)[None, :] * g_ref[0][None, :] \
        + be_ref[0][None, :]
    r = jnp.maximum(hb, 0.0)
    rw = jnp.dot(r, w2_ref[...], preferred_element_type=jnp.float32)
    y2_ref[...] = _pad_rows(rw * ns_ref[0][:, None])


def _tc_mid(p, nd, b1, gamma, beta, W2, ns):
    return pl.pallas_call(
        _tc_mid_body,
        out_shape=jax.ShapeDtypeStruct((NP, D), jnp.float32),
    )(p, nd, b1, gamma, beta, W2, ns)


def _tc_out_body(p_ref, nd_ref, b2_ref, o_ref):
    p = p_ref[...]
    o_ref[...] = (p[0, :N] + p[1, :N]) * nd_ref[0][:, None] \
        + b2_ref[0][None, :]


def _tc_out(p, nd, b2):
    return pl.pallas_call(
        _tc_out_body,
        out_shape=jax.ShapeDtypeStruct((N, D), jnp.float32),
    )(p, nd, b2)


@jax.jit
def _run(x, edge_index, W1, b1, gamma, beta, W2, b2):
    src = edge_index[0].astype(jnp.int32)
    dst = edge_index[1].astype(jnp.int32)

    idx2 = jnp.stack([src, dst]).reshape(2, NW, EPT)
    srcc = src.reshape(NW, EPT)
    dstc = dst.reshape(NW, NCHUNK, K)

    degp = _sc_degrees(idx2)
    xw = _tc_mm(x, W1)
    y1, ns, nd = _tc_scale(degp, xw)
    p1 = _sc_aggregate(y1, srcc, dstc)
    y2 = _tc_mid(p1, nd, b1.reshape(1, D), gamma.reshape(1, D),
                 beta.reshape(1, D), W2, ns)
    p2 = _sc_aggregate(y2, srcc, dstc)
    return _tc_out(p2, nd, b2.reshape(1, D))


def kernel(x, edge_index, W1, b1, gamma, beta, W2, b2):
    return _run(x, edge_index, W1, b1, gamma, beta, W2, b2)
